# Initial kernel scaffold; baseline (speedup 1.0000x reference)
import jax
import jax.numpy as jnp
from jax.experimental import pallas as pl


def kernel(moving_image, deformation_matrix):
    raise NotImplementedError("write your pallas kernel here")


# speedup vs baseline: 1.3668x; 1.3668x over previous
import functools

import jax
import jax.numpy as jnp
from jax import lax
from jax.experimental import pallas as pl
from jax.experimental.pallas import tpu as pltpu
from jax.experimental.pallas import tpu_sc as plsc

B, C, H, W, D = 2, 2, 128, 128, 128
Hp, Wp, Dp = H + 2, W + 2, D + 2
HWD = H * W * D
N_PTS = B * HWD
N_ROWS = B * Hp * Wp * Dp

NC, NS, L = 2, 16, 16
NW = NC * NS
PTS_W = N_PTS // NW
P = 2048
CHUNKS = PTS_W // P
VSTEPS = P // L


def _floor_i32(x):
    t = x.astype(jnp.int32)
    return jnp.where(t.astype(jnp.float32) > x, t - 1, t)


def _warp_body(im_hbm, dm_hbm, out_hbm,
               dxr, dyr, dzr, wr, o0r, o1r, sem, *idx_and_g):
    idxr = idx_and_g[:16]
    gr = idx_and_g[16:]
    wid = lax.axis_index("s") * NC + lax.axis_index("c")
    base_pt = wid * PTS_W
    b = base_pt // HWD
    dm_b = b * 3 * HWD
    out_b = b * C * HWD
    im_b = b * C * (Hp * Wp * Dp)

    def chunk(t, _):
        local0 = (base_pt % HWD) + t * P
        pltpu.sync_copy(dm_hbm.at[pl.ds(dm_b + 0 * HWD + local0, P)], dxr)
        pltpu.sync_copy(dm_hbm.at[pl.ds(dm_b + 1 * HWD + local0, P)], dyr)
        pltpu.sync_copy(dm_hbm.at[pl.ds(dm_b + 2 * HWD + local0, P)], dzr)

        iota = lax.iota(jnp.int32, L)

        def compute(v, _):
            sl = pl.ds(v * L, L)
            g = local0 + v * L + iota
            hh = (g >> 14) & 127
            ww = (g >> 7) & 127
            dd = g & 127
            x = dxr[sl] + hh.astype(jnp.float32) + 1.0
            y = dyr[sl] + ww.astype(jnp.float32) + 1.0
            z = dzr[sl] + dd.astype(jnp.float32) + 1.0
            x0f = _floor_i32(x)
            y0f = _floor_i32(y)
            z0f = _floor_i32(z)
            x0 = jnp.clip(x0f, 0, Hp - 1)
            x1 = jnp.clip(x0f + 1, 0, Hp - 1)
            y0 = jnp.clip(y0f, 0, Wp - 1)
            y1 = jnp.clip(y0f + 1, 0, Wp - 1)
            z0 = jnp.clip(z0f, 0, Dp - 1)
            z1 = jnp.clip(z0f + 1, 0, Dp - 1)
            ddx = x1.astype(jnp.float32) - x
            ddy = y1.astype(jnp.float32) - y
            ddz = z1.astype(jnp.float32) - z
            ex, ey, ez = 1.0 - ddx, 1.0 - ddy, 1.0 - ddz
            bx0 = im_b + x0 * (Wp * Dp)
            bx1 = im_b + x1 * (Wp * Dp)
            b00 = bx0 + y0 * Dp
            b01 = bx0 + y1 * Dp
            b10 = bx1 + y0 * Dp
            b11 = bx1 + y1 * Dp
            xy00 = ddx * ddy
            xy01 = ddx * ey
            xy10 = ex * ddy
            xy11 = ex * ey
            idxs = (b00 + z0, b00 + z1, b01 + z0, b01 + z1,
                    b10 + z0, b10 + z1, b11 + z0, b11 + z1)
            ws = (xy00 * ddz, xy00 * ez, xy01 * ddz, xy01 * ez,
                  xy10 * ddz, xy10 * ez, xy11 * ddz, xy11 * ez)
            for j in range(8):
                idxr[j][sl] = idxs[j]
                idxr[8 + j][sl] = idxs[j] + Hp * Wp * Dp
                wr[j, sl] = ws[j]
            return ()

        lax.fori_loop(0, VSTEPS, compute, (), unroll=False)

        copies = [pltpu.async_copy(im_hbm.at[idxr[j]], gr[j], sem)
                  for j in range(16)]
        for cp in copies:
            cp.wait()

        def combine(v, _):
            sl = pl.ds(v * L, L)
            acc0 = jnp.zeros((L,), jnp.float32)
            acc1 = jnp.zeros((L,), jnp.float32)
            for j in range(8):
                wj = wr[j, sl]
                acc0 = acc0 + wj * gr[j][sl]
                acc1 = acc1 + wj * gr[8 + j][sl]
            o0r[sl] = acc0
            o1r[sl] = acc1
            return ()

        lax.fori_loop(0, VSTEPS, combine, (), unroll=False)

        pltpu.sync_copy(o0r, out_hbm.at[pl.ds(out_b + 0 * HWD + local0, P)])
        pltpu.sync_copy(o1r, out_hbm.at[pl.ds(out_b + 1 * HWD + local0, P)])
        return ()

    lax.fori_loop(0, CHUNKS, chunk, (), unroll=False)


@jax.jit
def _warp(im_flat, dm_flat):
    mesh = plsc.VectorSubcoreMesh(core_axis_name="c", subcore_axis_name="s",
                                  num_cores=NC, num_subcores=NS)
    f = pl.kernel(
        _warp_body,
        out_type=jax.ShapeDtypeStruct((B * C * HWD,), jnp.float32),
        mesh=mesh,
        scratch_types=[
            pltpu.VMEM((P,), jnp.float32),
            pltpu.VMEM((P,), jnp.float32),
            pltpu.VMEM((P,), jnp.float32),
            pltpu.VMEM((8, P), jnp.float32),
            pltpu.VMEM((P,), jnp.float32),
            pltpu.VMEM((P,), jnp.float32),
            pltpu.SemaphoreType.DMA,
        ]
        + [pltpu.VMEM((P,), jnp.int32) for _ in range(16)]
        + [pltpu.VMEM((P,), jnp.float32) for _ in range(16)],
    )
    return f(im_flat, dm_flat)


def kernel(moving_image, deformation_matrix):
    assert moving_image.shape == (B, C, H, W, D)
    assert deformation_matrix.shape == (B, 3, H, W, D)
    im = jnp.pad(moving_image, ((0, 0), (0, 0), (1, 1), (1, 1), (1, 1)))
    im_flat = im.reshape(-1)
    dm_flat = deformation_matrix.reshape(-1)
    out = _warp(im_flat, dm_flat)
    return out.reshape(B, C, H, W, D)
